# baseline (speedup 1.0000x reference)
import jax
import jax.numpy as jnp
from jax import lax
from jax.experimental import pallas as pl
from jax.experimental.pallas import tpu as pltpu
from jax.experimental.pallas import tpu_sc as plsc

_THRESHOLD = 0.5
_PAD = -1
_L = 16

_B, _C = 4096, 1000
_NFULL = _C // _L
_LASTC = _C - _L
_SKIP = _NFULL * _L - _LASTC
_NC, _NS = 2, 16
_NW = _NC * _NS
_RPW = _B // _NW
_BLK = 8
_BPW = _RPW // _BLK


def _compact_row(ibuf, obuf, k):
  neg1 = jnp.full((_L,), _PAD, jnp.int32)
  iota = lax.iota(jnp.int32, _L)
  one = jnp.ones((_L,), jnp.int32)
  zero = jnp.zeros((_L,), jnp.int32)
  shifts = [jnp.where(iota >= s, iota - s, zero) for s in (1, 2, 4, 8)]
  keeps = [iota >= s for s in (1, 2, 4, 8)]

  for j in range(_NFULL):
    obuf[k, pl.ds(j * _L, _L)] = neg1
  obuf[k, pl.ds(_LASTC, _L)] = neg1

  rowv = zero + k
  cnt = zero - 1
  for c in range(_NFULL + 1):
    col = c * _L if c < _NFULL else _LASTC
    x = ibuf[k, pl.ds(col, _L)]
    m = x >= _THRESHOLD
    if c == _NFULL:
      m = m & (iota >= _SKIP)
    s = jnp.where(m, one, zero)
    for sh, keep in zip(shifts, keeps):
      g = s.at[sh].get(mode="promise_in_bounds")
      s = s + jnp.where(keep, g, zero)
    pos = cnt + s
    plsc.store_scatter(obuf, [rowv, pos], iota + col, mask=m)
    cnt = cnt + plsc.all_reduce_population_count(m)


def _body(probs_hbm, out_hbm,
          in0, in1, ob0, ob1,
          sem_in0, sem_in1, sem_out0, sem_out1):
  wid = lax.axis_index("s") * _NC + lax.axis_index("c")
  base = wid * _RPW

  in_bufs = (in0, in1)
  out_bufs = (ob0, ob1)
  sem_in = (sem_in0, sem_in1)
  sem_out = (sem_out0, sem_out1)

  for b in range(2):
    pltpu.async_copy(probs_hbm.at[pl.ds(base + b * _BLK, _BLK), :],
                     in_bufs[b], sem_in[b])

  def block_pair(i, carry):
    for b in range(2):
      blk = 2 * i + b
      r0 = base + blk * _BLK
      ibuf, obuf = in_bufs[b], out_bufs[b]

      pltpu.make_async_copy(probs_hbm.at[pl.ds(r0, _BLK), :],
                            ibuf, sem_in[b]).wait()

      @pl.when(blk >= 2)
      def _():
        pltpu.make_async_copy(obuf,
                              out_hbm.at[pl.ds(r0 - 2 * _BLK, _BLK), :],
                              sem_out[b]).wait()

      def rowfn(k, c2):
        _compact_row(ibuf, obuf, k)
        return c2

      lax.fori_loop(0, _BLK, rowfn, 0, unroll=False)

      pltpu.async_copy(obuf, out_hbm.at[pl.ds(r0, _BLK), :], sem_out[b])

      @pl.when(blk + 2 < _BPW)
      def _():
        pltpu.async_copy(probs_hbm.at[pl.ds(r0 + 2 * _BLK, _BLK), :],
                         ibuf, sem_in[b])
    return carry

  lax.fori_loop(0, _BPW // 2, block_pair, 0, unroll=False)

  for b in range(2):
    pltpu.make_async_copy(
        out_bufs[b],
        out_hbm.at[pl.ds(base + (_BPW - 2 + b) * _BLK, _BLK), :],
        sem_out[b]).wait()


def kernel(probs):
  return pl.kernel(
      _body,
      out_type=jax.ShapeDtypeStruct((_B, _C), jnp.int32),
      mesh=plsc.VectorSubcoreMesh(core_axis_name="c", subcore_axis_name="s"),
      compiler_params=pltpu.CompilerParams(
          needs_layout_passes=False, use_tc_tiling_on_sc=True),
      scratch_types=[
          pltpu.VMEM((_BLK, _C), jnp.float32),
          pltpu.VMEM((_BLK, _C), jnp.float32),
          pltpu.VMEM((_BLK, _C), jnp.int32),
          pltpu.VMEM((_BLK, _C), jnp.int32),
          pltpu.SemaphoreType.DMA,
          pltpu.SemaphoreType.DMA,
          pltpu.SemaphoreType.DMA,
          pltpu.SemaphoreType.DMA,
      ],
  )(probs)


# speedup vs baseline: 1.4263x; 1.4263x over previous
import jax
import jax.numpy as jnp
from jax import lax
from jax.experimental import pallas as pl
from jax.experimental.pallas import tpu as pltpu
from jax.experimental.pallas import tpu_sc as plsc

_THRESHOLD = 0.5
_PAD = -1
_L = 16

_B, _C = 4096, 1000
_NFULL = _C // _L
_LASTC = _C - _L
_SKIP = _NFULL * _L - _LASTC
_NC, _NS = 2, 16
_NW = _NC * _NS
_RPW = _B // _NW
_BLK = 16
_BPW = _RPW // _BLK


def _compact_row(ibuf, obuf, k):
  neg1 = jnp.full((_L,), _PAD, jnp.int32)
  iota = lax.iota(jnp.int32, _L)
  one = jnp.ones((_L,), jnp.int32)

  for j in range(_NFULL):
    obuf[k, pl.ds(j * _L, _L)] = neg1
  obuf[k, pl.ds(_LASTC, _L)] = neg1

  rowv = jnp.zeros((_L,), jnp.int32) + k
  cnt = jnp.zeros((_L,), jnp.int32) - 1
  for c in range(_NFULL + 1):
    col = c * _L if c < _NFULL else _LASTC
    x = ibuf[k, pl.ds(col, _L)]
    m = x >= _THRESHOLD
    if c == _NFULL:
      m = m & (iota >= _SKIP)
    pos = cnt + plsc.cumsum(one, mask=m)
    plsc.store_scatter(obuf, [rowv, pos], iota + col, mask=m)
    cnt = cnt + plsc.all_reduce_population_count(m)


def _body(probs_hbm, out_hbm,
          in0, in1, ob0, ob1,
          sem_in0, sem_in1, sem_out0, sem_out1):
  wid = lax.axis_index("s") * _NC + lax.axis_index("c")
  base = wid * _RPW

  in_bufs = (in0, in1)
  out_bufs = (ob0, ob1)
  sem_in = (sem_in0, sem_in1)
  sem_out = (sem_out0, sem_out1)

  for b in range(2):
    pltpu.async_copy(probs_hbm.at[pl.ds(base + b * _BLK, _BLK), :],
                     in_bufs[b], sem_in[b])

  def block_pair(i, carry):
    for b in range(2):
      blk = 2 * i + b
      r0 = base + blk * _BLK
      ibuf, obuf = in_bufs[b], out_bufs[b]

      pltpu.make_async_copy(probs_hbm.at[pl.ds(r0, _BLK), :],
                            ibuf, sem_in[b]).wait()

      @pl.when(blk >= 2)
      def _():
        pltpu.make_async_copy(obuf,
                              out_hbm.at[pl.ds(r0 - 2 * _BLK, _BLK), :],
                              sem_out[b]).wait()

      def rowfn(k, c2):
        _compact_row(ibuf, obuf, k)
        return c2

      lax.fori_loop(0, _BLK, rowfn, 0, unroll=False)

      pltpu.async_copy(obuf, out_hbm.at[pl.ds(r0, _BLK), :], sem_out[b])

      @pl.when(blk + 2 < _BPW)
      def _():
        pltpu.async_copy(probs_hbm.at[pl.ds(r0 + 2 * _BLK, _BLK), :],
                         ibuf, sem_in[b])
    return carry

  lax.fori_loop(0, _BPW // 2, block_pair, 0, unroll=False)

  for b in range(2):
    pltpu.make_async_copy(
        out_bufs[b],
        out_hbm.at[pl.ds(base + (_BPW - 2 + b) * _BLK, _BLK), :],
        sem_out[b]).wait()


def kernel(probs):
  return pl.kernel(
      _body,
      out_type=jax.ShapeDtypeStruct((_B, _C), jnp.int32),
      mesh=plsc.VectorSubcoreMesh(core_axis_name="c", subcore_axis_name="s"),
      compiler_params=pltpu.CompilerParams(
          needs_layout_passes=False, use_tc_tiling_on_sc=True),
      scratch_types=[
          pltpu.VMEM((_BLK, _C), jnp.float32),
          pltpu.VMEM((_BLK, _C), jnp.float32),
          pltpu.VMEM((_BLK, _C), jnp.int32),
          pltpu.VMEM((_BLK, _C), jnp.int32),
          pltpu.SemaphoreType.DMA,
          pltpu.SemaphoreType.DMA,
          pltpu.SemaphoreType.DMA,
          pltpu.SemaphoreType.DMA,
      ],
  )(probs)


# speedup vs baseline: 1.7158x; 1.2030x over previous
import jax
import jax.numpy as jnp
from jax import lax
from jax.experimental import pallas as pl
from jax.experimental.pallas import tpu as pltpu
from jax.experimental.pallas import tpu_sc as plsc

_THRESHOLD = 0.5
_PAD = -1
_L = 16

_B, _C = 4096, 1000
_NFULL = _C // _L
_LASTC = _C - _L
_SKIP = _NFULL * _L - _LASTC
_NC, _NS = 2, 16
_NW = _NC * _NS
_RPW = _B // _NW
_BLK = 8
_BPW = _RPW // _BLK

_TCR = 256
_SEG = 128


def _tc_pos_body(x_ref, o_ref):
  nseg = (_C + _SEG - 1) // _SEG
  base = jnp.zeros((_TCR, 1), jnp.float32)
  for t in range(nseg):
    lo = t * _SEG
    w = min(_SEG, _C - lo)
    seg = x_ref[:, lo:lo + w] >= _THRESHOLD
    segb = seg.astype(jnp.bfloat16)
    tri = (lax.broadcasted_iota(jnp.int32, (w, w), 0)
           <= lax.broadcasted_iota(jnp.int32, (w, w), 1)).astype(jnp.bfloat16)
    within = lax.dot_general(segb, tri, (((1,), (0,)), ((), ())),
                             preferred_element_type=jnp.float32)
    pos = (within + base).astype(jnp.int32) - 1
    o_ref[:, lo:lo + w] = jnp.where(seg, pos, _PAD)
    base = base + within[:, w - 1:w]


def _tc_pos(probs):
  return pl.pallas_call(
      _tc_pos_body,
      out_shape=jax.ShapeDtypeStruct((_B, _C), jnp.int32),
      grid=(_B // _TCR,),
      in_specs=[pl.BlockSpec((_TCR, _C), lambda i: (i, 0))],
      out_specs=pl.BlockSpec((_TCR, _C), lambda i: (i, 0)),
  )(probs)


def _scatter_row(ibuf, obuf, k):
  neg1 = jnp.full((_L,), _PAD, jnp.int32)
  iota = lax.iota(jnp.int32, _L)

  for j in range(_NFULL):
    obuf[k, pl.ds(j * _L, _L)] = neg1
  obuf[k, pl.ds(_LASTC, _L)] = neg1

  rowv = jnp.zeros((_L,), jnp.int32) + k
  for c in range(_NFULL + 1):
    col = c * _L if c < _NFULL else _LASTC
    posv = ibuf[k, pl.ds(col, _L)]
    m = posv >= 0
    if c == _NFULL:
      m = m & (iota >= _SKIP)
    plsc.store_scatter(obuf, [rowv, posv], iota + col, mask=m)


def _sc_body(pos_hbm, out_hbm,
             in0, in1, ob0, ob1,
             sem_in0, sem_in1, sem_out0, sem_out1):
  wid = lax.axis_index("s") * _NC + lax.axis_index("c")
  base = wid * _RPW

  in_bufs = (in0, in1)
  out_bufs = (ob0, ob1)
  sem_in = (sem_in0, sem_in1)
  sem_out = (sem_out0, sem_out1)

  for b in range(2):
    pltpu.async_copy(pos_hbm.at[pl.ds(base + b * _BLK, _BLK), :],
                     in_bufs[b], sem_in[b])

  def block_pair(i, carry):
    for b in range(2):
      blk = 2 * i + b
      r0 = base + blk * _BLK
      ibuf, obuf = in_bufs[b], out_bufs[b]

      pltpu.make_async_copy(pos_hbm.at[pl.ds(r0, _BLK), :],
                            ibuf, sem_in[b]).wait()

      @pl.when(blk >= 2)
      def _():
        pltpu.make_async_copy(obuf,
                              out_hbm.at[pl.ds(r0 - 2 * _BLK, _BLK), :],
                              sem_out[b]).wait()

      def rowfn(k, c2):
        _scatter_row(ibuf, obuf, k)
        return c2

      lax.fori_loop(0, _BLK, rowfn, 0, unroll=False)

      pltpu.async_copy(obuf, out_hbm.at[pl.ds(r0, _BLK), :], sem_out[b])

      @pl.when(blk + 2 < _BPW)
      def _():
        pltpu.async_copy(pos_hbm.at[pl.ds(r0 + 2 * _BLK, _BLK), :],
                         ibuf, sem_in[b])
    return carry

  lax.fori_loop(0, _BPW // 2, block_pair, 0, unroll=False)

  for b in range(2):
    pltpu.make_async_copy(
        out_bufs[b],
        out_hbm.at[pl.ds(base + (_BPW - 2 + b) * _BLK, _BLK), :],
        sem_out[b]).wait()


def _sc_scatter(pos):
  return pl.kernel(
      _sc_body,
      out_type=jax.ShapeDtypeStruct((_B, _C), jnp.int32),
      mesh=plsc.VectorSubcoreMesh(core_axis_name="c", subcore_axis_name="s"),
      compiler_params=pltpu.CompilerParams(
          needs_layout_passes=False, use_tc_tiling_on_sc=True),
      scratch_types=[
          pltpu.VMEM((_BLK, _C), jnp.int32),
          pltpu.VMEM((_BLK, _C), jnp.int32),
          pltpu.VMEM((_BLK, _C), jnp.int32),
          pltpu.VMEM((_BLK, _C), jnp.int32),
          pltpu.SemaphoreType.DMA,
          pltpu.SemaphoreType.DMA,
          pltpu.SemaphoreType.DMA,
          pltpu.SemaphoreType.DMA,
      ],
  )(pos)


def kernel(probs):
  return _sc_scatter(_tc_pos(probs))
